# Initial kernel scaffold; baseline (speedup 1.0000x reference)
import jax
import jax.numpy as jnp
from jax.experimental import pallas as pl


def kernel(input_ids, orig_table, new_table):
    raise NotImplementedError("write your pallas kernel here")


# speedup vs baseline: 3.5191x; 3.5191x over previous
import functools

import jax
import jax.numpy as jnp
from jax import lax
from jax.experimental import pallas as pl
from jax.experimental.pallas import tpu as pltpu
from jax.experimental.pallas import tpu_sc as plsc

ORIG_VOCAB = 1000000
NEW_VOCAB = 1000
D = 64
L = 16
NC, NS = 2, 16
NW = NC * NS
CHUNK = 128


def _sc_body(ids_hbm, orig_hbm, new_hbm, out_hbm, newtbl_v, idx_v, cid_v,
             rows_v, sem):
    wid = lax.axis_index("s") * NC + lax.axis_index("c")
    n = ids_hbm.shape[0]
    per_w = n // NW
    nchunks = per_w // CHUNK
    base_w = wid * per_w

    pltpu.sync_copy(new_hbm, newtbl_v)

    def chunk_body(ci, carry):
        base = base_w + ci * CHUNK
        pltpu.sync_copy(ids_hbm.at[pl.ds(base, CHUNK)], idx_v)
        for g in range(CHUNK // L):
            v = idx_v[pl.ds(g * L, L)]
            cid_v[pl.ds(g * L, L)] = jnp.minimum(v, ORIG_VOCAB - 1)
        pltpu.async_copy(orig_hbm.at[cid_v], rows_v, sem).wait()

        for g in range(CHUNK // L):
            v = idx_v[pl.ds(g * L, L)]
            m = v >= ORIG_VOCAB
            cnt = plsc.all_reduce_population_count(m)[0]

            @pl.when(cnt > 0)
            def _():
                nid = jnp.where(m, v - ORIG_VOCAB, 0)
                lrow = lax.iota(jnp.int32, L) + g * L
                for c in range(D):
                    col = jnp.full((L,), c, jnp.int32)
                    vals = plsc.load_gather(newtbl_v, [nid, col])
                    plsc.store_scatter(rows_v, [lrow, col], vals, mask=m)

        pltpu.sync_copy(rows_v, out_hbm.at[pl.ds(base, CHUNK)])
        return carry

    lax.fori_loop(0, nchunks, chunk_body, 0)


@functools.lru_cache(maxsize=None)
def _make_sc_call(n):
    mesh = plsc.VectorSubcoreMesh(core_axis_name="c", subcore_axis_name="s")
    return pl.kernel(
        _sc_body,
        out_type=jax.ShapeDtypeStruct((n, D), jnp.float32),
        mesh=mesh,
        scratch_types=[
            pltpu.VMEM((NEW_VOCAB, D), jnp.float32),
            pltpu.VMEM((CHUNK,), jnp.int32),
            pltpu.VMEM((CHUNK,), jnp.int32),
            pltpu.VMEM((CHUNK, D), jnp.float32),
            pltpu.SemaphoreType.DMA,
        ],
        compiler_params=pltpu.CompilerParams(
            use_tc_tiling_on_sc=False, needs_layout_passes=False),
    )


@jax.jit
def kernel(input_ids, orig_table, new_table):
    b, h = input_ids.shape
    ids = input_ids.reshape(-1).astype(jnp.int32)
    out = _make_sc_call(ids.shape[0])(ids, orig_table, new_table)
    return out.reshape(b, h, D)
